# Initial kernel scaffold; baseline (speedup 1.0000x reference)
import jax
import jax.numpy as jnp
from jax.experimental import pallas as pl


def kernel(preds_geom, targets_registration_vertices, nbs_idxs, nbs_weights):
    raise NotImplementedError("write your pallas kernel here")


# speedup vs baseline: 33.6168x; 33.6168x over previous
import functools

import jax
import jax.numpy as jnp
from jax import lax
from jax.experimental import pallas as pl
from jax.experimental.pallas import tpu as pltpu
from jax.experimental.pallas import tpu_sc as plsc

LANES = 16
NW = 32
NP = 102400
VT = NP // NW
VB = 128
SB = VT // VB
G = VB * 16 // 128


def _laplacian_sq_partials(d_rows, idx2, w_rows):
    mesh = plsc.VectorSubcoreMesh(core_axis_name="c", subcore_axis_name="s")

    @functools.partial(
        pl.kernel,
        mesh=mesh,
        out_type=jax.ShapeDtypeStruct((NW * 8, LANES), jnp.float32),
        compiler_params=pltpu.CompilerParams(use_tc_tiling_on_sc=False),
        scratch_types=[
            pltpu.VMEM((G, 128), jnp.int32),
            pltpu.VMEM((VB, LANES), jnp.float32),
            pltpu.VMEM((VB, LANES), jnp.float32),
            pltpu.VMEM((VB * 16, LANES), jnp.float32),
            pltpu.VMEM((8, LANES), jnp.float32),
            pltpu.SemaphoreType.DMA,
        ],
    )
    def k(d_hbm, idx_hbm, w_hbm, out_hbm, idx_v, w_v, down_v, rows_v, acc_v,
          sem):
        wid = lax.axis_index("s") * 2 + lax.axis_index("c")
        vbase0 = wid * VT

        def sub_block(sb, acc):
            vbase = pl.multiple_of(vbase0 + sb * VB, VB)
            ibase = pl.multiple_of(vbase * 16 // 128, 16)
            pltpu.sync_copy(idx_hbm.at[pl.ds(ibase, G)], idx_v)
            pltpu.sync_copy(w_hbm.at[pl.ds(vbase, VB)], w_v)
            pltpu.sync_copy(d_hbm.at[pl.ds(vbase, VB)], down_v)
            cps = []
            for g in range(G):
                cps.append(pltpu.async_copy(
                    d_hbm.at[idx_v.at[g]],
                    rows_v.at[pl.ds(g * 128, 128)], sem))
            for cp in cps:
                cp.wait()

            def vloop(v, acc):
                row = down_v[v, :]
                wrow = w_v[v, :]
                for kk in range(16):
                    row = row + wrow[kk] * rows_v[v * 16 + kk, :]
                return acc + row * row

            return lax.fori_loop(0, VB, vloop, acc)

        acc = lax.fori_loop(0, SB, sub_block,
                            jnp.zeros((LANES,), jnp.float32))
        acc_v[0, :] = acc
        zero = jnp.zeros((LANES,), jnp.float32)
        for r in range(1, 8):
            acc_v[r, :] = zero
        pltpu.sync_copy(acc_v, out_hbm.at[pl.ds(pl.multiple_of(wid * 8, 8),
                                                8)])

    return k(d_rows, idx2, w_rows)


def kernel(preds_geom, targets_registration_vertices, nbs_idxs, nbs_weights):
    B, N, D = preds_geom.shape
    K = nbs_idxs.shape[1]
    d = preds_geom - targets_registration_vertices
    d_rows = jnp.transpose(d, (1, 0, 2)).reshape(N, B * D)
    d_rows = jnp.pad(d_rows, ((0, NP - N), (0, LANES - B * D)))
    w_rows = jnp.pad(nbs_weights, ((0, NP - N), (0, LANES - K)))
    idx2 = jnp.pad(nbs_idxs, ((0, NP - N), (0, 0))).reshape(NP * K // 128,
                                                            128)
    partials = _laplacian_sq_partials(d_rows, idx2, w_rows)
    return jnp.sum(partials) / (B * N * D)
